# baseline (speedup 1.0000x reference)
import functools
import numpy as np
import jax
import jax.numpy as jnp
from jax import lax
from jax.experimental import pallas as pl
from jax.experimental.pallas import tpu as pltpu
from jax.experimental.pallas import tpu_sc as plsc

H = 16
HD = 8
NC = 2
NS = 16
NW = NC * NS
G = 128


def _ln_relu(x, g, b):
    m = jnp.mean(x, axis=-1, keepdims=True)
    v = jnp.mean((x - m) ** 2, axis=-1, keepdims=True)
    y = (x - m) * lax.rsqrt(v + 1e-5) * g + b
    return jnp.maximum(y, 0.0)


def _pack_bf16(a, b):
    au = lax.bitcast_convert_type(a.astype(jnp.bfloat16), jnp.uint16).astype(jnp.uint32)
    bu = lax.bitcast_convert_type(b.astype(jnp.bfloat16), jnp.uint16).astype(jnp.uint32)
    return lax.bitcast_convert_type(au | (bu << 16), jnp.float32)


def _unpack_lo(f):
    w = lax.bitcast_convert_type(f, jnp.uint32)
    return lax.bitcast_convert_type(w << 16, jnp.float32)


def _unpack_hi(f):
    w = lax.bitcast_convert_type(f, jnp.uint32)
    return lax.bitcast_convert_type(w & jnp.uint32(0xFFFF0000), jnp.float32)


def _tables_body(h_ref, wa_ref, wq2_ref, vq_ref, td_ref, ts_ref):
    h = h_ref[...]
    P = jnp.dot(h, wa_ref[...], preferred_element_type=jnp.float32)
    qpre = P[:, 512:640] + vq_ref[0:1, :]
    qr = _ln_relu(qpre, vq_ref[1:2, :], vq_ref[2:3, :])
    q = jnp.dot(qr, wq2_ref[...], preferred_element_type=jnp.float32) + vq_ref[3:4, :]
    td_ref[:, 0:128] = _pack_bf16(P[:, 0:128], P[:, 128:256])
    td_ref[:, 128:256] = q
    ts_ref[...] = _pack_bf16(P[:, 256:384], P[:, 384:512])


def _make_tables(h, wa, wq2, vq, bn):
    n = h.shape[0]
    grid = n // bn
    return pl.pallas_call(
        _tables_body,
        grid=(grid,),
        in_specs=[
            pl.BlockSpec((bn, 128), lambda i: (i, 0)),
            pl.BlockSpec((128, 640), lambda i: (0, 0)),
            pl.BlockSpec((128, 128), lambda i: (0, 0)),
            pl.BlockSpec((8, 128), lambda i: (0, 0)),
        ],
        out_specs=[
            pl.BlockSpec((bn, 256), lambda i: (i, 0)),
            pl.BlockSpec((bn, 128), lambda i: (i, 0)),
        ],
        out_shape=[
            jax.ShapeDtypeStruct((n, 256), jnp.float32),
            jax.ShapeDtypeStruct((n, 128), jnp.float32),
        ],
    )(h, wa, wq2, vq)


def _gather_body(nchunk, td, ts, didx, sidx, gd, gs,
                 idx_d, idx_s, rows_d, rows_s, sem_d0, sem_s0, sem_d1, sem_s1):
    wid = lax.axis_index("s") * NC + lax.axis_index("c")
    nloop = (nchunk + NW - 1) // NW
    assert nloop % 2 == 0
    sems = ((sem_d0, sem_s0), (sem_d1, sem_s1))

    def start(c, b):
        @pl.when(c < nchunk)
        def _():
            base = c * G
            pltpu.sync_copy(didx.at[pl.ds(base, G)], idx_d.at[b])
            pltpu.async_copy(td.at[idx_d.at[b]], rows_d.at[b], sems[b][0])
            pltpu.sync_copy(sidx.at[pl.ds(base, G)], idx_s.at[b])
            pltpu.async_copy(ts.at[idx_s.at[b]], rows_s.at[b], sems[b][1])

    def finish(c, b):
        @pl.when(c < nchunk)
        def _():
            base = c * G
            pltpu.make_async_copy(td.at[idx_d.at[b]], rows_d.at[b],
                                  sems[b][0]).wait()
            pltpu.make_async_copy(ts.at[idx_s.at[b]], rows_s.at[b],
                                  sems[b][1]).wait()
            pltpu.sync_copy(rows_d.at[b], gd.at[pl.ds(base, G)])
            pltpu.sync_copy(rows_s.at[b], gs.at[pl.ds(base, G)])

    start(wid, 0)

    def body(hh, carry):
        c0 = wid + NW * (2 * hh)
        c1 = wid + NW * (2 * hh + 1)
        c2 = wid + NW * (2 * hh + 2)
        start(c1, 1)
        finish(c0, 0)
        start(c2, 0)
        finish(c1, 1)
        return carry

    lax.fori_loop(0, nloop // 2, body, 0)


def _gather_rows(td, ts, dst, src):
    e = dst.shape[0]
    nchunk = e // G
    mesh = plsc.VectorSubcoreMesh(core_axis_name="c", subcore_axis_name="s")
    return pl.kernel(
        functools.partial(_gather_body, nchunk),
        out_type=[
            jax.ShapeDtypeStruct((e, 256), jnp.float32),
            jax.ShapeDtypeStruct((e, 128), jnp.float32),
        ],
        mesh=mesh,
        scratch_types=[
            pltpu.VMEM((2, G), jnp.int32),
            pltpu.VMEM((2, G), jnp.int32),
            pltpu.VMEM((2, G, 256), jnp.float32),
            pltpu.VMEM((2, G, 128), jnp.float32),
            pltpu.SemaphoreType.DMA,
            pltpu.SemaphoreType.DMA,
            pltpu.SemaphoreType.DMA,
            pltpu.SemaphoreType.DMA,
        ],
    )(td, ts, dst, src)


def _edge_body(ef_ref, rf_ref, gd_ref, gs_ref,
               wef_ref, wrf_ref, vecs_ref, w2_ref, mh_ref, mht_ref,
               ex_ref, evw_ref):
    P = (jnp.dot(ef_ref[...], wef_ref[...], preferred_element_type=jnp.float32)
         + jnp.dot(rf_ref[...], wrf_ref[...], preferred_element_type=jnp.float32))
    gdp = gd_ref[:, 0:128]
    gsp = gs_ref[...]
    pre_k = P[:, 0:128] + _unpack_lo(gdp) + _unpack_lo(gsp) + vecs_ref[0:1, :]
    rk = _ln_relu(pre_k, vecs_ref[1:2, :], vecs_ref[2:3, :])
    k = jnp.dot(rk, w2_ref[0], preferred_element_type=jnp.float32) + vecs_ref[3:4, :]
    pre_v = P[:, 128:256] + _unpack_hi(gdp) + _unpack_hi(gsp) + vecs_ref[4:5, :]
    rv = _ln_relu(pre_v, vecs_ref[5:6, :], vecs_ref[6:7, :])
    v = jnp.dot(rv, w2_ref[1], preferred_element_type=jnp.float32) + vecs_ref[7:8, :]
    ew = jax.nn.sigmoid(P[:, 256:257] + vecs_ref[8:9, 0:1])
    qd = gd_ref[:, 128:256]
    logits = jnp.dot(qd * k, mh_ref[...], preferred_element_type=jnp.float32)
    ex = jnp.exp(logits * (1.0 / np.sqrt(HD)))
    exb = jnp.dot(ex, mht_ref[...], preferred_element_type=jnp.float32)
    ex_ref[...] = exb
    evw_ref[...] = exb * v * ew


def _edge_stage(ef, rf, gdr, gsr, wef, wrf, vecs, w2, mh, mht, be):
    e = ef.shape[0]
    grid = e // be
    return pl.pallas_call(
        _edge_body,
        grid=(grid,),
        in_specs=[
            pl.BlockSpec((be, 16), lambda i: (i, 0)),
            pl.BlockSpec((be, 64), lambda i: (i, 0)),
            pl.BlockSpec((be, 256), lambda i: (i, 0)),
            pl.BlockSpec((be, 128), lambda i: (i, 0)),
            pl.BlockSpec((16, 384), lambda i: (0, 0)),
            pl.BlockSpec((64, 384), lambda i: (0, 0)),
            pl.BlockSpec((16, 128), lambda i: (0, 0)),
            pl.BlockSpec((2, 128, 128), lambda i: (0, 0, 0)),
            pl.BlockSpec((128, 16), lambda i: (0, 0)),
            pl.BlockSpec((16, 128), lambda i: (0, 0)),
        ],
        out_specs=[
            pl.BlockSpec((be, 128), lambda i: (i, 0)),
            pl.BlockSpec((be, 128), lambda i: (i, 0)),
        ],
        out_shape=[
            jax.ShapeDtypeStruct((e, 128), jnp.float32),
            jax.ShapeDtypeStruct((e, 128), jnp.float32),
        ],
    )(ef, rf, gdr, gsr, wef, wrf, vecs, w2, mh, mht)


def _scatter_body(nchunk, n, evw_hbm, exb_hbm, didx, z128, accv, accd,
                  idx_v, r128, acc_sh, sem_i0, sem_p0, sem_i1, sem_p1):
    cid = lax.axis_index("c")
    sid = lax.axis_index("s")
    nloop = (nchunk + NS - 1) // NS

    @pl.when(sid == 0)
    def _():
        pltpu.sync_copy(z128, acc_sh)

    plsc.subcore_barrier()
    assert nloop % 2 == 0
    sems = ((sem_i0, sem_p0), (sem_i1, sem_p1))

    def start(c, b):
        @pl.when(c < nchunk)
        def _():
            base = c * G
            pltpu.async_copy(didx.at[pl.ds(base, G)], idx_v.at[b], sems[b][0])

            @pl.when(cid == 0)
            def _():
                pltpu.async_copy(evw_hbm.at[pl.ds(base, G)], r128.at[b],
                                 sems[b][1])

            @pl.when(cid == 1)
            def _():
                pltpu.async_copy(exb_hbm.at[pl.ds(base, G)], r128.at[b],
                                 sems[b][1])

    def finish(c, b):
        @pl.when(c < nchunk)
        def _():
            base = c * G
            pltpu.make_async_copy(didx.at[pl.ds(base, G)], idx_v.at[b],
                                  sems[b][0]).wait()
            pltpu.make_async_copy(evw_hbm.at[pl.ds(base, G)], r128.at[b],
                                  sems[b][1]).wait()
            pltpu.sync_copy(r128.at[b], acc_sh.at[idx_v.at[b]], add=True)

    start(sid, 0)

    def body(hh, carry):
        c0 = sid + NS * (2 * hh)
        c1 = sid + NS * (2 * hh + 1)
        c2 = sid + NS * (2 * hh + 2)
        start(c1, 1)
        finish(c0, 0)
        start(c2, 0)
        finish(c1, 1)
        return carry

    lax.fori_loop(0, nloop // 2, body, 0)
    plsc.subcore_barrier()

    rows = (n // NS) // 8 * 8
    tail = n - NS * rows

    @pl.when(cid == 0)
    def _():
        pltpu.sync_copy(acc_sh.at[pl.ds(sid * rows, rows)],
                        accv.at[pl.ds(sid * rows, rows)])

    @pl.when(cid == 1)
    def _():
        pltpu.sync_copy(acc_sh.at[pl.ds(sid * rows, rows)],
                        accd.at[pl.ds(sid * rows, rows)])

    if tail:
        @pl.when((sid == NS - 1) & (cid == 0))
        def _():
            pltpu.sync_copy(acc_sh.at[pl.ds(NS * rows, tail)],
                            accv.at[pl.ds(NS * rows, tail)])

        @pl.when((sid == NS - 1) & (cid == 1))
        def _():
            pltpu.sync_copy(acc_sh.at[pl.ds(NS * rows, tail)],
                            accd.at[pl.ds(NS * rows, tail)])


def _scatter_rows(evw, exb, dst, n):
    e = dst.shape[0]
    nchunk = e // G
    z128 = jnp.zeros((n + 8, 128), jnp.float32)
    mesh = plsc.VectorSubcoreMesh(core_axis_name="c", subcore_axis_name="s")
    return pl.kernel(
        functools.partial(_scatter_body, nchunk, n),
        out_type=[
            jax.ShapeDtypeStruct((n, 128), jnp.float32),
            jax.ShapeDtypeStruct((n, 128), jnp.float32),
        ],
        mesh=mesh,
        scratch_types=[
            pltpu.VMEM((2, G), jnp.int32),
            pltpu.VMEM((2, G, 128), jnp.float32),
            pltpu.VMEM_SHARED((n + 8, 128), jnp.float32),
            pltpu.SemaphoreType.DMA,
            pltpu.SemaphoreType.DMA,
            pltpu.SemaphoreType.DMA,
            pltpu.SemaphoreType.DMA,
        ],
    )(evw, exb, dst, z128)


def _final_body(*refs):
    acc_refs, (h_ref, wn1_ref, wn2_ref, vn_ref, out_ref) = refs[:-5], refs[-5:]
    sv = acc_refs[0][...]
    denb = acc_refs[1][...]
    for i in range(2, len(acc_refs), 2):
        sv = sv + acc_refs[i][...]
        denb = denb + acc_refs[i + 1][...]
    aggr = sv / (denb + 1e-16)
    h = h_ref[...]
    pre = (jnp.dot(aggr, wn1_ref[0], preferred_element_type=jnp.float32)
           + jnp.dot(h, wn1_ref[1], preferred_element_type=jnp.float32)
           + vn_ref[0:1, :])
    r = _ln_relu(pre, vn_ref[1:2, :], vn_ref[2:3, :])
    out = jnp.dot(r, wn2_ref[...], preferred_element_type=jnp.float32) + vn_ref[3:4, :]
    out_ref[...] = out + h


def _final_stage(accs, h, wn1, wn2, vn, bn):
    n = h.shape[0]
    grid = n // bn
    return pl.pallas_call(
        _final_body,
        grid=(grid,),
        in_specs=(
            [pl.BlockSpec((bn, 128), lambda i: (i, 0))] * (len(accs) + 1)
            + [
                pl.BlockSpec((2, 128, 128), lambda i: (0, 0, 0)),
                pl.BlockSpec((128, 128), lambda i: (0, 0)),
                pl.BlockSpec((8, 128), lambda i: (0, 0)),
            ]
        ),
        out_specs=pl.BlockSpec((bn, 128), lambda i: (i, 0)),
        out_shape=jax.ShapeDtypeStruct((n, 128), jnp.float32),
    )(*accs, h, wn1, wn2, vn)


def kernel(h, r_feat, edge_feat, edge_index, params):
    n, d = h.shape
    pk, pv, pq, pn = params["hk"], params["hv"], params["hq"], params["node"]
    src = edge_index[0]
    dst = edge_index[1]

    wa = jnp.concatenate([pk["W1"][80:208], pv["W1"][80:208],
                          pk["W1"][208:336], pv["W1"][208:336], pq["W1"]], axis=1)
    vq = jnp.stack([pq["b1"], pq["g"], pq["be"], pq["b2"],
                    jnp.zeros_like(pq["b1"]), jnp.zeros_like(pq["b1"]),
                    jnp.zeros_like(pq["b1"]), jnp.zeros_like(pq["b1"])])

    wef = jnp.pad(jnp.concatenate([pk["W1"][0:16], pv["W1"][0:16]], axis=1),
                  ((0, 0), (0, 128)))
    wrf = jnp.pad(jnp.concatenate([pk["W1"][16:80], pv["W1"][16:80],
                                   params["ew_W"]], axis=1), ((0, 0), (0, 127)))
    ewb = jnp.zeros((128,), jnp.float32).at[0].set(params["ew_b"][0])
    vecs = jnp.concatenate([
        jnp.stack([pk["b1"], pk["g"], pk["be"], pk["b2"],
                   pv["b1"], pv["g"], pv["be"], pv["b2"]]),
        jnp.stack([ewb] + [jnp.zeros((128,), jnp.float32)] * 7)])
    w2 = jnp.stack([pk["W2"], pv["W2"]])

    mh = jnp.asarray(np.repeat(np.eye(H, dtype=np.float32), HD, axis=0))
    mht = jnp.asarray(np.repeat(np.eye(H, dtype=np.float32), HD, axis=1))

    td, ts = _make_tables(h, wa, pq["W2"], vq, 1000)

    e = dst.shape[0]
    npiece = 4
    eh = (e // npiece) // 640 * 640
    bounds = [i * eh for i in range(npiece)] + [e]
    accs = []
    for lo, hi in zip(bounds[:-1], bounds[1:]):
        sl = slice(lo, hi)
        gdr, gsr = _gather_rows(td, ts, dst[sl], src[sl])
        exb, evw = _edge_stage(edge_feat[sl], r_feat[sl], gdr, gsr, wef, wrf,
                               vecs, w2, mh, mht, 5000)
        accs.extend(_scatter_rows(evw, exb, dst[sl], n))

    wn1 = jnp.stack([pn["W1"][0:128], pn["W1"][128:256]])
    vn = jnp.stack([pn["b1"], pn["g"], pn["be"], pn["b2"],
                    jnp.zeros_like(pn["b1"]), jnp.zeros_like(pn["b1"]),
                    jnp.zeros_like(pn["b1"]), jnp.zeros_like(pn["b1"])])
    return _final_stage(accs, h, wn1, pn["W2"], vn, 1000)


# speedup vs baseline: 1.0215x; 1.0215x over previous
import functools
import numpy as np
import jax
import jax.numpy as jnp
from jax import lax
from jax.experimental import pallas as pl
from jax.experimental.pallas import tpu as pltpu
from jax.experimental.pallas import tpu_sc as plsc

H = 16
HD = 8
NC = 2
NS = 16
NW = NC * NS
G = 128


def _ln_relu(x, g, b):
    m = jnp.mean(x, axis=-1, keepdims=True)
    v = jnp.mean((x - m) ** 2, axis=-1, keepdims=True)
    y = (x - m) * lax.rsqrt(v + 1e-5) * g + b
    return jnp.maximum(y, 0.0)


def _pack_bf16(a, b):
    au = lax.bitcast_convert_type(a.astype(jnp.bfloat16), jnp.uint16).astype(jnp.uint32)
    bu = lax.bitcast_convert_type(b.astype(jnp.bfloat16), jnp.uint16).astype(jnp.uint32)
    return lax.bitcast_convert_type(au | (bu << 16), jnp.float32)


def _unpack_lo(f):
    w = lax.bitcast_convert_type(f, jnp.uint32)
    return lax.bitcast_convert_type(w << 16, jnp.float32)


def _unpack_hi(f):
    w = lax.bitcast_convert_type(f, jnp.uint32)
    return lax.bitcast_convert_type(w & jnp.uint32(0xFFFF0000), jnp.float32)


def _tables_body(h_ref, wa_ref, wq2_ref, vq_ref, td_ref, ts_ref):
    h = h_ref[...]
    P = jnp.dot(h, wa_ref[...], preferred_element_type=jnp.float32)
    qpre = P[:, 512:640] + vq_ref[0:1, :]
    qr = _ln_relu(qpre, vq_ref[1:2, :], vq_ref[2:3, :])
    q = jnp.dot(qr, wq2_ref[...], preferred_element_type=jnp.float32) + vq_ref[3:4, :]
    td_ref[:, 0:128] = _pack_bf16(P[:, 0:128], P[:, 128:256])
    td_ref[:, 128:256] = q
    ts_ref[...] = _pack_bf16(P[:, 256:384], P[:, 384:512])


def _make_tables(h, wa, wq2, vq, bn):
    n = h.shape[0]
    grid = n // bn
    return pl.pallas_call(
        _tables_body,
        grid=(grid,),
        in_specs=[
            pl.BlockSpec((bn, 128), lambda i: (i, 0)),
            pl.BlockSpec((128, 640), lambda i: (0, 0)),
            pl.BlockSpec((128, 128), lambda i: (0, 0)),
            pl.BlockSpec((8, 128), lambda i: (0, 0)),
        ],
        out_specs=[
            pl.BlockSpec((bn, 256), lambda i: (i, 0)),
            pl.BlockSpec((bn, 128), lambda i: (i, 0)),
        ],
        out_shape=[
            jax.ShapeDtypeStruct((n, 256), jnp.float32),
            jax.ShapeDtypeStruct((n, 128), jnp.float32),
        ],
    )(h, wa, wq2, vq)


def _gather_body(nchunk, td, ts, didx, sidx, gd, gs,
                 idx_d, idx_s, rows_d, rows_s, sem_d0, sem_s0, sem_d1, sem_s1):
    wid = lax.axis_index("s") * NC + lax.axis_index("c")
    nloop = (nchunk + NW - 1) // NW
    assert nloop % 2 == 0
    sems = ((sem_d0, sem_s0), (sem_d1, sem_s1))

    def start(c, b):
        @pl.when(c < nchunk)
        def _():
            base = c * G
            pltpu.sync_copy(didx.at[pl.ds(base, G)], idx_d.at[b])
            pltpu.async_copy(td.at[idx_d.at[b]], rows_d.at[b], sems[b][0])
            pltpu.sync_copy(sidx.at[pl.ds(base, G)], idx_s.at[b])
            pltpu.async_copy(ts.at[idx_s.at[b]], rows_s.at[b], sems[b][1])

    def finish(c, b):
        @pl.when(c < nchunk)
        def _():
            base = c * G
            pltpu.make_async_copy(td.at[idx_d.at[b]], rows_d.at[b],
                                  sems[b][0]).wait()
            pltpu.make_async_copy(ts.at[idx_s.at[b]], rows_s.at[b],
                                  sems[b][1]).wait()
            pltpu.sync_copy(rows_d.at[b], gd.at[pl.ds(base, G)])
            pltpu.sync_copy(rows_s.at[b], gs.at[pl.ds(base, G)])

    start(wid, 0)

    def body(hh, carry):
        c0 = wid + NW * (2 * hh)
        c1 = wid + NW * (2 * hh + 1)
        c2 = wid + NW * (2 * hh + 2)
        start(c1, 1)
        finish(c0, 0)
        start(c2, 0)
        finish(c1, 1)
        return carry

    lax.fori_loop(0, nloop // 2, body, 0)


def _gather_rows(td, ts, dst, src):
    e = dst.shape[0]
    nchunk = e // G
    mesh = plsc.VectorSubcoreMesh(core_axis_name="c", subcore_axis_name="s")
    return pl.kernel(
        functools.partial(_gather_body, nchunk),
        out_type=[
            jax.ShapeDtypeStruct((e, 256), jnp.float32),
            jax.ShapeDtypeStruct((e, 128), jnp.float32),
        ],
        mesh=mesh,
        scratch_types=[
            pltpu.VMEM((2, G), jnp.int32),
            pltpu.VMEM((2, G), jnp.int32),
            pltpu.VMEM((2, G, 256), jnp.float32),
            pltpu.VMEM((2, G, 128), jnp.float32),
            pltpu.SemaphoreType.DMA,
            pltpu.SemaphoreType.DMA,
            pltpu.SemaphoreType.DMA,
            pltpu.SemaphoreType.DMA,
        ],
    )(td, ts, dst, src)


def _edge_body(ef_ref, rf_ref, gd_ref, gs_ref,
               wef_ref, wrf_ref, vecs_ref, w2_ref, mh_ref, mht_ref,
               ex_ref, evw_ref):
    P = (jnp.dot(ef_ref[...], wef_ref[...], preferred_element_type=jnp.float32)
         + jnp.dot(rf_ref[...], wrf_ref[...], preferred_element_type=jnp.float32))
    gdp = gd_ref[:, 0:128]
    gsp = gs_ref[...]
    pre_k = P[:, 0:128] + _unpack_lo(gdp) + _unpack_lo(gsp) + vecs_ref[0:1, :]
    rk = _ln_relu(pre_k, vecs_ref[1:2, :], vecs_ref[2:3, :])
    k = jnp.dot(rk, w2_ref[0], preferred_element_type=jnp.float32) + vecs_ref[3:4, :]
    pre_v = P[:, 128:256] + _unpack_hi(gdp) + _unpack_hi(gsp) + vecs_ref[4:5, :]
    rv = _ln_relu(pre_v, vecs_ref[5:6, :], vecs_ref[6:7, :])
    v = jnp.dot(rv, w2_ref[1], preferred_element_type=jnp.float32) + vecs_ref[7:8, :]
    ew = jax.nn.sigmoid(P[:, 256:257] + vecs_ref[8:9, 0:1])
    qd = gd_ref[:, 128:256]
    logits = jnp.dot(qd * k, mh_ref[...], preferred_element_type=jnp.float32)
    ex = jnp.exp(logits * (1.0 / np.sqrt(HD)))
    exb = jnp.dot(ex, mht_ref[...], preferred_element_type=jnp.float32)
    ex_ref[...] = exb
    evw_ref[...] = exb * v * ew


def _edge_stage(ef, rf, gdr, gsr, wef, wrf, vecs, w2, mh, mht, be):
    e = ef.shape[0]
    grid = e // be
    return pl.pallas_call(
        _edge_body,
        grid=(grid,),
        in_specs=[
            pl.BlockSpec((be, 16), lambda i: (i, 0)),
            pl.BlockSpec((be, 64), lambda i: (i, 0)),
            pl.BlockSpec((be, 256), lambda i: (i, 0)),
            pl.BlockSpec((be, 128), lambda i: (i, 0)),
            pl.BlockSpec((16, 384), lambda i: (0, 0)),
            pl.BlockSpec((64, 384), lambda i: (0, 0)),
            pl.BlockSpec((16, 128), lambda i: (0, 0)),
            pl.BlockSpec((2, 128, 128), lambda i: (0, 0, 0)),
            pl.BlockSpec((128, 16), lambda i: (0, 0)),
            pl.BlockSpec((16, 128), lambda i: (0, 0)),
        ],
        out_specs=[
            pl.BlockSpec((be, 128), lambda i: (i, 0)),
            pl.BlockSpec((be, 128), lambda i: (i, 0)),
        ],
        out_shape=[
            jax.ShapeDtypeStruct((e, 128), jnp.float32),
            jax.ShapeDtypeStruct((e, 128), jnp.float32),
        ],
    )(ef, rf, gdr, gsr, wef, wrf, vecs, w2, mh, mht)


def _scatter_body(nchunk, n, evw_hbm, exb_hbm, didx, z128, accv, accd,
                  idx_v, r128, acc_sh, sem_i0, sem_p0, sem_i1, sem_p1):
    cid = lax.axis_index("c")
    sid = lax.axis_index("s")
    nloop = (nchunk + NS - 1) // NS

    @pl.when(sid == 0)
    def _():
        pltpu.sync_copy(z128, acc_sh)

    plsc.subcore_barrier()
    assert nloop % 2 == 0
    sems = ((sem_i0, sem_p0), (sem_i1, sem_p1))

    def start(c, b):
        @pl.when(c < nchunk)
        def _():
            base = c * G
            pltpu.async_copy(didx.at[pl.ds(base, G)], idx_v.at[b], sems[b][0])

            @pl.when(cid == 0)
            def _():
                pltpu.async_copy(evw_hbm.at[pl.ds(base, G)], r128.at[b],
                                 sems[b][1])

            @pl.when(cid == 1)
            def _():
                pltpu.async_copy(exb_hbm.at[pl.ds(base, G)], r128.at[b],
                                 sems[b][1])

    def finish(c, b):
        @pl.when(c < nchunk)
        def _():
            base = c * G
            pltpu.make_async_copy(didx.at[pl.ds(base, G)], idx_v.at[b],
                                  sems[b][0]).wait()
            pltpu.make_async_copy(evw_hbm.at[pl.ds(base, G)], r128.at[b],
                                  sems[b][1]).wait()
            pltpu.sync_copy(r128.at[b], acc_sh.at[idx_v.at[b]], add=True)

    start(sid, 0)

    def body(hh, carry):
        c0 = sid + NS * (2 * hh)
        c1 = sid + NS * (2 * hh + 1)
        c2 = sid + NS * (2 * hh + 2)
        start(c1, 1)
        finish(c0, 0)
        start(c2, 0)
        finish(c1, 1)
        return carry

    lax.fori_loop(0, nloop // 2, body, 0)
    plsc.subcore_barrier()

    rows = (n // NS) // 8 * 8
    tail = n - NS * rows

    @pl.when(cid == 0)
    def _():
        pltpu.sync_copy(acc_sh.at[pl.ds(sid * rows, rows)],
                        accv.at[pl.ds(sid * rows, rows)])

    @pl.when(cid == 1)
    def _():
        pltpu.sync_copy(acc_sh.at[pl.ds(sid * rows, rows)],
                        accd.at[pl.ds(sid * rows, rows)])

    if tail:
        @pl.when((sid == NS - 1) & (cid == 0))
        def _():
            pltpu.sync_copy(acc_sh.at[pl.ds(NS * rows, tail)],
                            accv.at[pl.ds(NS * rows, tail)])

        @pl.when((sid == NS - 1) & (cid == 1))
        def _():
            pltpu.sync_copy(acc_sh.at[pl.ds(NS * rows, tail)],
                            accd.at[pl.ds(NS * rows, tail)])


def _scatter_rows(evw, exb, dst, n):
    e = dst.shape[0]
    nchunk = e // G
    z128 = jnp.zeros((n + 8, 128), jnp.float32)
    mesh = plsc.VectorSubcoreMesh(core_axis_name="c", subcore_axis_name="s")
    return pl.kernel(
        functools.partial(_scatter_body, nchunk, n),
        out_type=[
            jax.ShapeDtypeStruct((n, 128), jnp.float32),
            jax.ShapeDtypeStruct((n, 128), jnp.float32),
        ],
        mesh=mesh,
        scratch_types=[
            pltpu.VMEM((2, G), jnp.int32),
            pltpu.VMEM((2, G, 128), jnp.float32),
            pltpu.VMEM_SHARED((n + 8, 128), jnp.float32),
            pltpu.SemaphoreType.DMA,
            pltpu.SemaphoreType.DMA,
            pltpu.SemaphoreType.DMA,
            pltpu.SemaphoreType.DMA,
        ],
    )(evw, exb, dst, z128)


def _final_body(*refs):
    acc_refs, (h_ref, wn1_ref, wn2_ref, vn_ref, out_ref) = refs[:-5], refs[-5:]
    sv = acc_refs[0][...]
    denb = acc_refs[1][...]
    for i in range(2, len(acc_refs), 2):
        sv = sv + acc_refs[i][...]
        denb = denb + acc_refs[i + 1][...]
    aggr = sv / (denb + 1e-16)
    h = h_ref[...]
    pre = (jnp.dot(aggr, wn1_ref[0], preferred_element_type=jnp.float32)
           + jnp.dot(h, wn1_ref[1], preferred_element_type=jnp.float32)
           + vn_ref[0:1, :])
    r = _ln_relu(pre, vn_ref[1:2, :], vn_ref[2:3, :])
    out = jnp.dot(r, wn2_ref[...], preferred_element_type=jnp.float32) + vn_ref[3:4, :]
    out_ref[...] = out + h


def _final_stage(accs, h, wn1, wn2, vn, bn):
    n = h.shape[0]
    grid = n // bn
    return pl.pallas_call(
        _final_body,
        grid=(grid,),
        in_specs=(
            [pl.BlockSpec((bn, 128), lambda i: (i, 0))] * (len(accs) + 1)
            + [
                pl.BlockSpec((2, 128, 128), lambda i: (0, 0, 0)),
                pl.BlockSpec((128, 128), lambda i: (0, 0)),
                pl.BlockSpec((8, 128), lambda i: (0, 0)),
            ]
        ),
        out_specs=pl.BlockSpec((bn, 128), lambda i: (i, 0)),
        out_shape=jax.ShapeDtypeStruct((n, 128), jnp.float32),
    )(*accs, h, wn1, wn2, vn)


def kernel(h, r_feat, edge_feat, edge_index, params):
    n, d = h.shape
    pk, pv, pq, pn = params["hk"], params["hv"], params["hq"], params["node"]
    src = edge_index[0]
    dst = edge_index[1]

    wa = jnp.concatenate([pk["W1"][80:208], pv["W1"][80:208],
                          pk["W1"][208:336], pv["W1"][208:336], pq["W1"]], axis=1)
    vq = jnp.stack([pq["b1"], pq["g"], pq["be"], pq["b2"],
                    jnp.zeros_like(pq["b1"]), jnp.zeros_like(pq["b1"]),
                    jnp.zeros_like(pq["b1"]), jnp.zeros_like(pq["b1"])])

    wef = jnp.pad(jnp.concatenate([pk["W1"][0:16], pv["W1"][0:16]], axis=1),
                  ((0, 0), (0, 128)))
    wrf = jnp.pad(jnp.concatenate([pk["W1"][16:80], pv["W1"][16:80],
                                   params["ew_W"]], axis=1), ((0, 0), (0, 127)))
    ewb = jnp.zeros((128,), jnp.float32).at[0].set(params["ew_b"][0])
    vecs = jnp.concatenate([
        jnp.stack([pk["b1"], pk["g"], pk["be"], pk["b2"],
                   pv["b1"], pv["g"], pv["be"], pv["b2"]]),
        jnp.stack([ewb] + [jnp.zeros((128,), jnp.float32)] * 7)])
    w2 = jnp.stack([pk["W2"], pv["W2"]])

    mh = jnp.asarray(np.repeat(np.eye(H, dtype=np.float32), HD, axis=0))
    mht = jnp.asarray(np.repeat(np.eye(H, dtype=np.float32), HD, axis=1))

    td, ts = _make_tables(h, wa, pq["W2"], vq, 1000)

    e = dst.shape[0]
    npiece = 4
    eh = (e // npiece) // 640 * 640
    bounds = [i * eh for i in range(npiece)] + [e]
    accs = []
    for lo, hi in zip(bounds[:-1], bounds[1:]):
        sl = slice(lo, hi)
        be = max(b for b in (4000, 2000, 640, 512, 128) if (hi - lo) % b == 0)
        gdr, gsr = _gather_rows(td, ts, dst[sl], src[sl])
        exb, evw = _edge_stage(edge_feat[sl], r_feat[sl], gdr, gsr, wef, wrf,
                               vecs, w2, mh, mht, be)
        accs.extend(_scatter_rows(evw, exb, dst[sl], n))

    wn1 = jnp.stack([pn["W1"][0:128], pn["W1"][128:256]])
    vn = jnp.stack([pn["b1"], pn["g"], pn["be"], pn["b2"],
                    jnp.zeros_like(pn["b1"]), jnp.zeros_like(pn["b1"]),
                    jnp.zeros_like(pn["b1"]), jnp.zeros_like(pn["b1"])])
    return _final_stage(accs, h, wn1, pn["W2"], vn, 1000)
